# baseline (speedup 1.0000x reference)
import functools

import jax
import jax.numpy as jnp
from jax import lax
from jax.experimental import pallas as pl
from jax.experimental.pallas import tpu as pltpu
from jax.experimental.pallas import tpu_sc as plsc

N_POINTS = 10000
NUM_FEATURES = 128
NUM_CLASSES = 16
K_NN = 16
EPS_BN = 1e-5


def _head_body(x_ref, w1_ref, b1_ref, g1_ref, be1_ref, w2_ref, b2_ref, o_ref):
    x = x_ref[...]
    h = lax.dot_general(x.astype(jnp.bfloat16), w1_ref[...].astype(jnp.bfloat16),
                        (((1,), (1,)), ((), ())),
                        preferred_element_type=jnp.float32) + b1_ref[...]
    m = jnp.mean(h, axis=0)
    v = jnp.mean((h - m) ** 2, axis=0)
    h = (h - m) / jnp.sqrt(v + EPS_BN) * g1_ref[...] + be1_ref[...]
    h = jnp.maximum(h, 0.0)
    o = lax.dot_general(h.astype(jnp.bfloat16), w2_ref[...].astype(jnp.bfloat16),
                        (((1,), (1,)), ((), ())),
                        preferred_element_type=jnp.float32) + b2_ref[...]
    shifted = o - jnp.max(o, axis=-1, keepdims=True)
    o_ref[...] = shifted - jnp.log(jnp.sum(jnp.exp(shifted), axis=-1, keepdims=True))


def _head_pallas(xcat, p1, p2):
    n = xcat.shape[0]
    return pl.pallas_call(
        _head_body,
        out_shape=jax.ShapeDtypeStruct((n, NUM_CLASSES), jnp.float32),
    )(xcat, p1['w'], p1['b'], p1['g'], p1['be'], p2['w'], p2['b'])


def _fps_body(m, n, r, planes_ref, prow_ref, out_ref):
    fio = (lax.broadcasted_iota(jnp.int32, (r, 128), 0) * 128
           + lax.broadcasted_iota(jnp.int32, (r, 128), 1))
    px = planes_ref[0]
    py = planes_ref[1]
    pz = planes_ref[2]
    dist0 = jnp.where(fio < n, jnp.inf, -jnp.inf).astype(jnp.float32)
    out_ref[pl.ds(0, 1), :] = jnp.zeros((1, 1), jnp.int32)

    def step(i, carry):
        dist, last = carry
        prow = prow_ref[pl.ds(last, 1), :]
        lx, ly, lz = prow[0, 0], prow[0, 1], prow[0, 2]
        dx, dy, dz = px - lx, py - ly, pz - lz
        d = (dx * dx + dy * dy) + dz * dz
        dist = jnp.minimum(dist, d)
        mx = jnp.max(dist)
        idx = jnp.min(jnp.where(dist == mx, fio, jnp.int32(2**30)))
        out_ref[pl.ds(i, 1), :] = jnp.full((1, 1), idx, jnp.int32)
        return dist, idx

    lax.fori_loop(1, m, step, (dist0, jnp.int32(0)), unroll=False)


def _fps_pallas(pos, ratio):
    n = pos.shape[0]
    m = int(n * ratio)
    p = ((n + 127) // 128) * 128
    r = p // 128
    planes = jnp.pad(pos, ((0, p - n), (0, 0))).T.reshape(3, r, 128)
    prow = jnp.pad(pos, ((0, p - n), (0, 125)))
    out = pl.pallas_call(
        functools.partial(_fps_body, m, n, r),
        out_shape=jax.ShapeDtypeStruct((m, 1), jnp.int32),
    )(planes, prow)
    return out[:, 0]


def _knn_body(n_q, n_k, p, ch, k, excl, planes_ref, q_ref, out_ref):
    i = pl.program_id(0)
    px, py, pz = planes_ref[0], planes_ref[1], planes_ref[2]
    qx, qy, qz = q_ref[:, 0:1], q_ref[:, 1:2], q_ref[:, 2:3]

    def _rb(v):
        return v.astype(jnp.bfloat16).astype(jnp.float32)

    dot = (_rb(qx) * _rb(px) + _rb(qy) * _rb(py)) + _rb(qz) * _rb(pz)
    qsq = (qx * qx + qy * qy) + qz * qz
    sq = (px * px + py * py) + pz * pz
    d = (qsq - 2.0 * dot) + sq
    colio = lax.broadcasted_iota(jnp.int32, (ch, p), 1)
    if excl:
        rowio = lax.broadcasted_iota(jnp.int32, (ch, p), 0) + i * ch
        d = jnp.where(colio == rowio, jnp.inf, d)
    d = jnp.where(colio >= n_k, jnp.inf, d)
    outs = []
    for _ in range(k):
        mn = jnp.min(d, axis=1, keepdims=True)
        sel = jnp.min(jnp.where(d == mn, colio, jnp.int32(2**30)), axis=1)
        outs.append(sel[:, None])
        d = jnp.where(colio == sel[:, None], jnp.inf, d)
    out_ref[...] = jnp.concatenate(outs, axis=1)


def _knn_topk(query, keys, k, exclude_self):
    n_q, n_k = query.shape[0], keys.shape[0]
    p = ((n_k + 127) // 128) * 128
    qpad = ((n_q + 7) // 8) * 8
    ch = qpad if qpad <= 512 else 512
    qpad = ((n_q + ch - 1) // ch) * ch
    planes = jnp.pad(keys, ((0, p - n_k), (0, 0))).T.reshape(3, 1, p)
    qrows = jnp.pad(query, ((0, qpad - n_q), (0, 0)))
    out = pl.pallas_call(
        functools.partial(_knn_body, n_q, n_k, p, ch, k, exclude_self),
        grid=(qpad // ch,),
        in_specs=[
            pl.BlockSpec((3, 1, p), lambda i: (0, 0, 0)),
            pl.BlockSpec((ch, 3), lambda i: (i, 0)),
        ],
        out_specs=pl.BlockSpec((ch, k), lambda i: (i, 0)),
        out_shape=jax.ShapeDtypeStruct((qpad, k), jnp.int32),
    )(planes, qrows)
    return out[:n_q]


def _sc_gather(table, idx):
    v_rows, d = table.shape
    b = idx.shape[0]
    ch = 64 if d > 256 else 128
    chunks = b // (32 * ch)
    npairs = chunks // 2
    mesh = plsc.VectorSubcoreMesh(core_axis_name="c", subcore_axis_name="s")

    @functools.partial(
        pl.kernel, mesh=mesh,
        out_type=jax.ShapeDtypeStruct((b, d), jnp.float32),
        scratch_types=[
            pltpu.VMEM((ch,), jnp.int32),
            pltpu.VMEM((ch,), jnp.int32),
            pltpu.VMEM((ch, d), jnp.float32),
            pltpu.VMEM((ch, d), jnp.float32),
            pltpu.SemaphoreType.DMA,
            pltpu.SemaphoreType.DMA,
        ],
    )
    def k(table_hbm, idx_hbm, out_hbm, idx0, idx1, r0, r1, s0, s1):
        wid = lax.axis_index("s") * 2 + lax.axis_index("c")
        first = wid * chunks * ch
        pltpu.sync_copy(idx_hbm.at[pl.ds(first, ch)], idx0)
        pltpu.async_copy(table_hbm.at[idx0], r0, s0)

        def body(p, carry):
            b0 = (wid * chunks + 2 * p) * ch
            b1 = b0 + ch
            pltpu.sync_copy(idx_hbm.at[pl.ds(b1, ch)], idx1)
            pltpu.async_copy(table_hbm.at[idx1], r1, s1)
            pltpu.make_async_copy(table_hbm.at[idx0], r0, s0).wait()
            pltpu.sync_copy(r0, out_hbm.at[pl.ds(b0, ch)])

            @pl.when(p + 1 < npairs)
            def _():
                b2 = b1 + ch
                pltpu.sync_copy(idx_hbm.at[pl.ds(b2, ch)], idx0)
                pltpu.async_copy(table_hbm.at[idx0], r0, s0)

            pltpu.make_async_copy(table_hbm.at[idx1], r1, s1).wait()
            pltpu.sync_copy(r1, out_hbm.at[pl.ds(b1, ch)])
            return carry

        lax.fori_loop(0, npairs, body, 0)

    return k(table, idx)


def _gather_rows(table, idx, d_pad=None):
    v_rows, d = table.shape
    b = idx.shape[0]
    dp = d_pad if d_pad is not None else ((d + 127) // 128) * 128
    unit = 32 * 2 * (64 if dp > 256 else 128)
    bp = ((b + unit - 1) // unit) * unit
    t = table if dp == d else jnp.pad(table, ((0, 0), (0, dp - d)))
    ix = jnp.pad(idx, (0, bp - b)) if bp != b else idx
    out = _sc_gather(t, ix)
    return out[:b, :d]


def _econv_a_body(n, rx, xr_ref, xc_ref, w1_ref, b1_ref, h1_ref, st_ref):
    i = pl.program_id(0)
    cin = xc_ref.shape[1]
    xr = xr_ref[...]
    xc = xc_ref[...]
    xcr = jnp.broadcast_to(xc[:, None, :], (rx, K_NN, cin)).reshape(rx * K_NN, cin)
    feat = jnp.concatenate([xr, xcr - xr], axis=1)
    h = lax.dot_general(feat.astype(jnp.bfloat16), w1_ref[...].astype(jnp.bfloat16),
                        (((1,), (1,)), ((), ())),
                        preferred_element_type=jnp.float32) + b1_ref[...]
    h1_ref[...] = h
    rowio = lax.broadcasted_iota(jnp.int32, (rx * K_NN, 1), 0) // K_NN + i * rx
    hm = jnp.where(rowio < n, h, 0.0)
    st_ref[...] = jnp.concatenate(
        [jnp.sum(hm, axis=0)[None, None, :], jnp.sum(hm * hm, axis=0)[None, None, :]],
        axis=1)


def _econv_b_body(n, rx, h1_ref, m_ref, v_ref, g_ref, be_ref, w2_ref, b2_ref,
                  h2_ref, st_ref):
    i = pl.program_id(0)
    h = h1_ref[...]
    hb = (h - m_ref[...]) / jnp.sqrt(v_ref[...] + EPS_BN) * g_ref[...] + be_ref[...]
    gact = jnp.maximum(hb, 0.0)
    h2 = lax.dot_general(gact.astype(jnp.bfloat16), w2_ref[...].astype(jnp.bfloat16),
                         (((1,), (1,)), ((), ())),
                         preferred_element_type=jnp.float32) + b2_ref[...]
    h2_ref[...] = h2
    rowio = lax.broadcasted_iota(jnp.int32, (rx * K_NN, 1), 0) // K_NN + i * rx
    hm = jnp.where(rowio < n, h2, 0.0)
    st_ref[...] = jnp.concatenate(
        [jnp.sum(hm, axis=0)[None, None, :], jnp.sum(hm * hm, axis=0)[None, None, :]],
        axis=1)


def _econv_c_body(h2_ref, m_ref, v_ref, g_ref, be_ref, o_ref):
    h = h2_ref[...]
    hb = (h - m_ref[...]) / jnp.sqrt(v_ref[...] + EPS_BN) * g_ref[...] + be_ref[...]
    o_ref[...] = jnp.maximum(hb, 0.0)


def _edge_conv_pallas(p, x, pos, k=K_NN):
    n, cin = x.shape
    cout = p['w1'].shape[0]
    nbr = _knn_topk(lax.stop_gradient(pos), lax.stop_gradient(pos), k, True)

    if n == 10000:
        rx = 400
    else:
        rx = min(512, ((n + 7) // 8) * 8)
    n_pad = ((n + rx - 1) // rx) * rx
    nch = n_pad // rx
    e_tc = n_pad * k
    x_pad = jnp.pad(x, ((0, n_pad - n), (0, 0))) if n_pad != n else x
    row = nbr.reshape(-1)
    row_pad = jnp.pad(row, (0, e_tc - n * k))

    xr = _gather_rows(x_pad, row_pad)

    blk = rx * k
    h1, st1 = pl.pallas_call(
        functools.partial(_econv_a_body, n, rx),
        grid=(nch,),
        in_specs=[
            pl.BlockSpec((blk, cin), lambda i: (i, 0)),
            pl.BlockSpec((rx, cin), lambda i: (i, 0)),
            pl.BlockSpec((cout, 2 * cin), lambda i: (0, 0)),
            pl.BlockSpec((1, cout), lambda i: (0, 0)),
        ],
        out_specs=[
            pl.BlockSpec((blk, cout), lambda i: (i, 0)),
            pl.BlockSpec((1, 2, cout), lambda i: (i, 0, 0)),
        ],
        out_shape=[
            jax.ShapeDtypeStruct((e_tc, cout), jnp.float32),
            jax.ShapeDtypeStruct((nch, 2, cout), jnp.float32),
        ],
    )(xr, x_pad, p['w1'], p['b1'][None, :])
    e_real = jnp.float32(n * k)
    s = jnp.sum(st1, axis=0)
    m1 = (s[0] / e_real)[None, :]
    v1 = (s[1] / e_real - (s[0] / e_real) ** 2)[None, :]

    h2, st2 = pl.pallas_call(
        functools.partial(_econv_b_body, n, rx),
        grid=(nch,),
        in_specs=[
            pl.BlockSpec((blk, cout), lambda i: (i, 0)),
            pl.BlockSpec((1, cout), lambda i: (0, 0)),
            pl.BlockSpec((1, cout), lambda i: (0, 0)),
            pl.BlockSpec((1, cout), lambda i: (0, 0)),
            pl.BlockSpec((1, cout), lambda i: (0, 0)),
            pl.BlockSpec((cout, cout), lambda i: (0, 0)),
            pl.BlockSpec((1, cout), lambda i: (0, 0)),
        ],
        out_specs=[
            pl.BlockSpec((blk, cout), lambda i: (i, 0)),
            pl.BlockSpec((1, 2, cout), lambda i: (i, 0, 0)),
        ],
        out_shape=[
            jax.ShapeDtypeStruct((e_tc, cout), jnp.float32),
            jax.ShapeDtypeStruct((nch, 2, cout), jnp.float32),
        ],
    )(h1, m1, v1, p['g1'][None, :], p['be1'][None, :], p['w2'], p['b2'][None, :])
    s2 = jnp.sum(st2, axis=0)
    m2 = (s2[0] / e_real)[None, :]
    v2 = (s2[1] / e_real - (s2[0] / e_real) ** 2)[None, :]

    h3 = pl.pallas_call(
        _econv_c_body,
        grid=(nch,),
        in_specs=[
            pl.BlockSpec((blk, cout), lambda i: (i, 0)),
            pl.BlockSpec((1, cout), lambda i: (0, 0)),
            pl.BlockSpec((1, cout), lambda i: (0, 0)),
            pl.BlockSpec((1, cout), lambda i: (0, 0)),
            pl.BlockSpec((1, cout), lambda i: (0, 0)),
        ],
        out_specs=pl.BlockSpec((blk, cout), lambda i: (i, 0)),
        out_shape=jax.ShapeDtypeStruct((e_tc, cout), jnp.float32),
    )(h2, m2, v2, p['g2'][None, :], p['be2'][None, :])

    agg = jnp.zeros((n, cout), jnp.float32).at[row].max(h3[:n * k])
    return agg


def _lbr_body(x_ref, w_ref, b_ref, g_ref, be_ref, o_ref):
    h = lax.dot_general(x_ref[...].astype(jnp.bfloat16), w_ref[...].astype(jnp.bfloat16),
                        (((1,), (1,)), ((), ())),
                        preferred_element_type=jnp.float32) + b_ref[...]
    m = jnp.mean(h, axis=0)
    v = jnp.mean((h - m) ** 2, axis=0)
    hb = (h - m) / jnp.sqrt(v + EPS_BN) * g_ref[...] + be_ref[...]
    o_ref[...] = jnp.maximum(hb, 0.0)


def _lbr_pallas(p, x):
    n = x.shape[0]
    cout = p['w'].shape[0]
    return pl.pallas_call(
        _lbr_body,
        out_shape=jax.ShapeDtypeStruct((n, cout), jnp.float32),
    )(x, p['w'], p['b'][None, :], p['g'][None, :], p['be'][None, :])


def _wsum_body(xg_ref, w_ref, o_ref):
    r = w_ref.shape[0]
    c = xg_ref.shape[1]
    xg = xg_ref[...].reshape(r, 3, c)
    w = w_ref[...]
    o_ref[...] = ((xg[:, 0, :] * w[:, 0:1] + xg[:, 1, :] * w[:, 1:2])
                  + xg[:, 2, :] * w[:, 2:3])


def _knn_interpolate_pallas(x, pos_x, pos_y, k=3):
    idx = _knn_topk(lax.stop_gradient(pos_y), lax.stop_gradient(pos_x), k, False)
    diff = pos_y[:, None, :] - pos_x[idx]
    sqd = jnp.sum(diff * diff, axis=-1)
    w = 1.0 / jnp.clip(sqd, 1e-16, None)
    w = w / jnp.sum(w, axis=1, keepdims=True)
    n = pos_y.shape[0]
    c = x.shape[1]
    xg = _gather_rows(x, idx.reshape(-1))
    rb = 512
    n_pad = ((n + rb - 1) // rb) * rb
    xg = jnp.pad(xg, ((0, (n_pad - n) * 3), (0, 0)))
    w = jnp.pad(w, ((0, n_pad - n), (0, 0)))
    out = pl.pallas_call(
        _wsum_body,
        grid=(n_pad // rb,),
        in_specs=[
            pl.BlockSpec((3 * rb, c), lambda i: (i, 0)),
            pl.BlockSpec((rb, 3), lambda i: (i, 0)),
        ],
        out_specs=pl.BlockSpec((rb, c), lambda i: (i, 0)),
        out_shape=jax.ShapeDtypeStruct((n_pad, c), jnp.float32),
    )(xg, w)
    return out[:n]


def _batchnorm(x, g, b):
    m = jnp.mean(x, axis=0)
    v = jnp.var(x, axis=0)
    return (x - m) / jnp.sqrt(v + EPS_BN) * g + b


def _knn_graph(pos, k, chunk=2500):
    n = pos.shape[0]
    sq = jnp.sum(pos * pos, axis=1)
    nbrs = []
    for i in range(0, n, chunk):
        q = pos[i:i + chunk]
        d = jnp.sum(q * q, axis=1, keepdims=True) - 2.0 * (q @ pos.T) + sq[None, :]
        gidx = jnp.arange(i, i + q.shape[0])
        d = jnp.where(jnp.arange(n)[None, :] == gidx[:, None], jnp.inf, d)
        _, idx = jax.lax.top_k(-d, k)
        nbrs.append(idx)
    nbr = jnp.concatenate(nbrs, axis=0)
    row = nbr.reshape(-1)
    col = jnp.repeat(jnp.arange(n), k)
    return row, col


def _knn_query(query, keys, k, chunk=2500):
    ksq = jnp.sum(keys * keys, axis=1)
    out = []
    for i in range(0, query.shape[0], chunk):
        q = query[i:i + chunk]
        d = jnp.sum(q * q, axis=1, keepdims=True) - 2.0 * (q @ keys.T) + ksq[None, :]
        _, idx = jax.lax.top_k(-d, k)
        out.append(idx)
    return jnp.concatenate(out, axis=0)


def _edge_conv(p, x, pos, k=K_NN):
    nbr = _knn_topk(lax.stop_gradient(pos), lax.stop_gradient(pos), k, True)
    row = nbr.reshape(-1)
    col = jnp.repeat(jnp.arange(pos.shape[0]), k)
    feat = jnp.concatenate([x[row], x[col] - x[row]], axis=1)
    h = feat @ p['w1'].T + p['b1']
    h = jax.nn.relu(_batchnorm(h, p['g1'], p['be1']))
    h = h @ p['w2'].T + p['b2']
    h = jax.nn.relu(_batchnorm(h, p['g2'], p['be2']))
    agg = jnp.zeros((x.shape[0], h.shape[1]), h.dtype).at[row].max(h)
    return agg


def _fps(pos, ratio):
    n = pos.shape[0]
    m = int(n * ratio)
    pos = lax.stop_gradient(pos)

    def body(i, state):
        dist, idxs = state
        last = idxs[i - 1]
        d = jnp.sum((pos - pos[last]) ** 2, axis=1)
        dist = jnp.minimum(dist, d)
        idxs = idxs.at[i].set(jnp.argmax(dist).astype(jnp.int32))
        return dist, idxs

    dist0 = jnp.full((n,), jnp.inf, jnp.float32)
    idxs0 = jnp.zeros((m,), jnp.int32)
    _, idxs = lax.fori_loop(1, m, body, (dist0, idxs0))
    return idxs


def _knn_interpolate(x, pos_x, pos_y, k=3):
    idx = _knn_topk(lax.stop_gradient(pos_y), lax.stop_gradient(pos_x), k, False)
    diff = pos_y[:, None, :] - pos_x[idx]
    sqd = jnp.sum(diff * diff, axis=-1)
    w = 1.0 / jnp.clip(sqd, 1e-16, None)
    w = w / jnp.sum(w, axis=1, keepdims=True)
    return jnp.sum(x[idx] * w[..., None], axis=1)


def _lbr(p, x):
    h = x @ p['w'].T + p['b']
    return jax.nn.relu(_batchnorm(h, p['g'], p['be']))


def kernel(x, pos, batch, params):
    x0, pos0 = x, pos
    x1 = _edge_conv_pallas(params['conv1'], x0, pos0)
    idx1 = _fps_pallas(pos0, 0.25)
    pos1, x1s = pos0[idx1], x1[idx1]
    x2 = _edge_conv_pallas(params['conv2'], x1s, pos1)
    idx2 = _fps_pallas(pos1, 0.25)
    pos2, x2s = pos1[idx2], x2[idx2]
    x3 = _edge_conv_pallas(params['conv3'], x2s, pos2)
    idx3 = _fps_pallas(pos2, 0.25)
    pos3, x3s = pos2[idx3], x3[idx3]
    x4 = _edge_conv_pallas(params['conv4'], x3s, pos3)
    up2 = _knn_interpolate_pallas(x4, pos3, pos2)
    d2 = _lbr_pallas(params['dec1'], jnp.concatenate([up2, x3], axis=1))
    up1 = _knn_interpolate_pallas(d2, pos2, pos1)
    d1 = _lbr_pallas(params['dec2'], jnp.concatenate([up1, x2], axis=1))
    up0 = _knn_interpolate_pallas(d1, pos1, pos0)
    d0 = _lbr_pallas(params['dec3'], jnp.concatenate([up0, x1], axis=1))
    final = jnp.concatenate([d0, x0], axis=1)
    return _head_pallas(final, params['head1'], params['head2'])


# speedup vs baseline: 1.0169x; 1.0169x over previous
import functools

import jax
import jax.numpy as jnp
from jax import lax
from jax.experimental import pallas as pl
from jax.experimental.pallas import tpu as pltpu
from jax.experimental.pallas import tpu_sc as plsc

N_POINTS = 10000
NUM_FEATURES = 128
NUM_CLASSES = 16
K_NN = 16
EPS_BN = 1e-5


def _head_body(x_ref, w1_ref, b1_ref, g1_ref, be1_ref, w2_ref, b2_ref, o_ref):
    x = x_ref[...]
    h = lax.dot_general(x.astype(jnp.bfloat16), w1_ref[...].astype(jnp.bfloat16),
                        (((1,), (1,)), ((), ())),
                        preferred_element_type=jnp.float32) + b1_ref[...]
    m = jnp.mean(h, axis=0)
    v = jnp.mean((h - m) ** 2, axis=0)
    h = (h - m) / jnp.sqrt(v + EPS_BN) * g1_ref[...] + be1_ref[...]
    h = jnp.maximum(h, 0.0)
    o = lax.dot_general(h.astype(jnp.bfloat16), w2_ref[...].astype(jnp.bfloat16),
                        (((1,), (1,)), ((), ())),
                        preferred_element_type=jnp.float32) + b2_ref[...]
    shifted = o - jnp.max(o, axis=-1, keepdims=True)
    o_ref[...] = shifted - jnp.log(jnp.sum(jnp.exp(shifted), axis=-1, keepdims=True))


def _head_pallas(xcat, p1, p2):
    n = xcat.shape[0]
    return pl.pallas_call(
        _head_body,
        out_shape=jax.ShapeDtypeStruct((n, NUM_CLASSES), jnp.float32),
    )(xcat, p1['w'], p1['b'], p1['g'], p1['be'], p2['w'], p2['b'])


def _fps_body(m, n, r, planes_ref, prow_ref, out_ref):
    fio = (lax.broadcasted_iota(jnp.int32, (r, 128), 0) * 128
           + lax.broadcasted_iota(jnp.int32, (r, 128), 1))
    px = planes_ref[0]
    py = planes_ref[1]
    pz = planes_ref[2]
    dist0 = jnp.where(fio < n, jnp.inf, -jnp.inf).astype(jnp.float32)
    out_ref[pl.ds(0, 1), :] = jnp.zeros((1, 1), jnp.int32)

    def step(i, carry):
        dist, last = carry
        prow = prow_ref[pl.ds(last, 1), :]
        lx, ly, lz = prow[0, 0], prow[0, 1], prow[0, 2]
        dx, dy, dz = px - lx, py - ly, pz - lz
        d = (dx * dx + dy * dy) + dz * dz
        dist = jnp.minimum(dist, d)
        mx = jnp.max(dist)
        idx = jnp.min(jnp.where(dist == mx, fio, jnp.int32(2**30)))
        out_ref[pl.ds(i, 1), :] = jnp.full((1, 1), idx, jnp.int32)
        return dist, idx

    lax.fori_loop(1, m, step, (dist0, jnp.int32(0)), unroll=False)


def _fps_pallas(pos, ratio):
    n = pos.shape[0]
    m = int(n * ratio)
    p = ((n + 127) // 128) * 128
    r = p // 128
    planes = jnp.pad(pos, ((0, p - n), (0, 0))).T.reshape(3, r, 128)
    prow = jnp.pad(pos, ((0, p - n), (0, 125)))
    out = pl.pallas_call(
        functools.partial(_fps_body, m, n, r),
        out_shape=jax.ShapeDtypeStruct((m, 1), jnp.int32),
    )(planes, prow)
    return out[:, 0]


def _knn_body(n_q, n_k, p, ch, k, excl, planes_ref, q_ref, out_ref):
    i = pl.program_id(0)
    px, py, pz = planes_ref[0], planes_ref[1], planes_ref[2]
    qx, qy, qz = q_ref[:, 0:1], q_ref[:, 1:2], q_ref[:, 2:3]

    def _rb(v):
        return v.astype(jnp.bfloat16).astype(jnp.float32)

    dot = (_rb(qx) * _rb(px) + _rb(qy) * _rb(py)) + _rb(qz) * _rb(pz)
    qsq = (qx * qx + qy * qy) + qz * qz
    sq = (px * px + py * py) + pz * pz
    d = (qsq - 2.0 * dot) + sq
    colio = lax.broadcasted_iota(jnp.int32, (ch, p), 1)
    if excl:
        rowio = lax.broadcasted_iota(jnp.int32, (ch, p), 0) + i * ch
        d = jnp.where(colio == rowio, jnp.inf, d)
    d = jnp.where(colio >= n_k, jnp.inf, d)
    outs = []
    for _ in range(k):
        mn = jnp.min(d, axis=1, keepdims=True)
        sel = jnp.min(jnp.where(d == mn, colio, jnp.int32(2**30)), axis=1)
        outs.append(sel[:, None])
        d = jnp.where(colio == sel[:, None], jnp.inf, d)
    out_ref[...] = jnp.concatenate(outs, axis=1)


def _knn_topk(query, keys, k, exclude_self):
    n_q, n_k = query.shape[0], keys.shape[0]
    p = ((n_k + 127) // 128) * 128
    qpad = ((n_q + 7) // 8) * 8
    ch = qpad if qpad <= 512 else 512
    qpad = ((n_q + ch - 1) // ch) * ch
    planes = jnp.pad(keys, ((0, p - n_k), (0, 0))).T.reshape(3, 1, p)
    qrows = jnp.pad(query, ((0, qpad - n_q), (0, 0)))
    out = pl.pallas_call(
        functools.partial(_knn_body, n_q, n_k, p, ch, k, exclude_self),
        grid=(qpad // ch,),
        in_specs=[
            pl.BlockSpec((3, 1, p), lambda i: (0, 0, 0)),
            pl.BlockSpec((ch, 3), lambda i: (i, 0)),
        ],
        out_specs=pl.BlockSpec((ch, k), lambda i: (i, 0)),
        out_shape=jax.ShapeDtypeStruct((qpad, k), jnp.int32),
    )(planes, qrows)
    return out[:n_q]


def _sc_gather(table, idx):
    v_rows, d = table.shape
    b = idx.shape[0]
    ch = 128
    chunks = b // (32 * ch)
    mesh = plsc.VectorSubcoreMesh(core_axis_name="c", subcore_axis_name="s")

    @functools.partial(
        pl.kernel, mesh=mesh,
        out_type=jax.ShapeDtypeStruct((b, d), jnp.float32),
        scratch_types=[
            pltpu.VMEM((ch,), jnp.int32),
            pltpu.VMEM((ch, d), jnp.float32),
            pltpu.SemaphoreType.DMA,
        ],
    )
    def k(table_hbm, idx_hbm, out_hbm, idx_v, rows_v, sem):
        wid = lax.axis_index("s") * 2 + lax.axis_index("c")

        def body(c, carry):
            base = (wid * chunks + c) * ch
            pltpu.sync_copy(idx_hbm.at[pl.ds(base, ch)], idx_v)
            pltpu.async_copy(table_hbm.at[idx_v], rows_v, sem).wait()
            pltpu.sync_copy(rows_v, out_hbm.at[pl.ds(base, ch)])
            return carry

        lax.fori_loop(0, chunks, body, 0)

    return k(table, idx)


def _gather_rows(table, idx, d_pad=None):
    v_rows, d = table.shape
    b = idx.shape[0]
    dp = d_pad if d_pad is not None else ((d + 127) // 128) * 128
    bp = ((b + 4095) // 4096) * 4096
    t = table if dp == d else jnp.pad(table, ((0, 0), (0, dp - d)))
    ix = jnp.pad(idx, (0, bp - b)) if bp != b else idx
    out = _sc_gather(t, ix)
    return out[:b, :d]


def _econv_a_body(n, rx, xr_ref, xc_ref, w1_ref, b1_ref, h1_ref, st_ref):
    i = pl.program_id(0)
    cin = xc_ref.shape[1]
    xr = xr_ref[...]
    xc = xc_ref[...]
    xcr = jnp.broadcast_to(xc[:, None, :], (rx, K_NN, cin)).reshape(rx * K_NN, cin)
    feat = jnp.concatenate([xr, xcr - xr], axis=1)
    h = lax.dot_general(feat.astype(jnp.bfloat16), w1_ref[...].astype(jnp.bfloat16),
                        (((1,), (1,)), ((), ())),
                        preferred_element_type=jnp.float32) + b1_ref[...]
    h1_ref[...] = h
    rowio = lax.broadcasted_iota(jnp.int32, (rx * K_NN, 1), 0) // K_NN + i * rx
    hm = jnp.where(rowio < n, h, 0.0)
    st_ref[...] = jnp.concatenate(
        [jnp.sum(hm, axis=0)[None, None, :], jnp.sum(hm * hm, axis=0)[None, None, :]],
        axis=1)


def _econv_b_body(n, rx, h1_ref, m_ref, v_ref, g_ref, be_ref, w2_ref, b2_ref,
                  h2_ref, st_ref):
    i = pl.program_id(0)
    h = h1_ref[...]
    hb = (h - m_ref[...]) / jnp.sqrt(v_ref[...] + EPS_BN) * g_ref[...] + be_ref[...]
    gact = jnp.maximum(hb, 0.0)
    h2 = lax.dot_general(gact.astype(jnp.bfloat16), w2_ref[...].astype(jnp.bfloat16),
                         (((1,), (1,)), ((), ())),
                         preferred_element_type=jnp.float32) + b2_ref[...]
    h2_ref[...] = h2
    rowio = lax.broadcasted_iota(jnp.int32, (rx * K_NN, 1), 0) // K_NN + i * rx
    hm = jnp.where(rowio < n, h2, 0.0)
    st_ref[...] = jnp.concatenate(
        [jnp.sum(hm, axis=0)[None, None, :], jnp.sum(hm * hm, axis=0)[None, None, :]],
        axis=1)


def _econv_c_body(h2_ref, m_ref, v_ref, g_ref, be_ref, o_ref):
    h = h2_ref[...]
    hb = (h - m_ref[...]) / jnp.sqrt(v_ref[...] + EPS_BN) * g_ref[...] + be_ref[...]
    o_ref[...] = jnp.maximum(hb, 0.0)


def _edge_conv_pallas(p, x, pos, k=K_NN):
    n, cin = x.shape
    cout = p['w1'].shape[0]
    nbr = _knn_topk(lax.stop_gradient(pos), lax.stop_gradient(pos), k, True)

    if n == 10000:
        rx = 400
    else:
        rx = min(512, ((n + 7) // 8) * 8)
    n_pad = ((n + rx - 1) // rx) * rx
    nch = n_pad // rx
    e_tc = n_pad * k
    x_pad = jnp.pad(x, ((0, n_pad - n), (0, 0))) if n_pad != n else x
    row = nbr.reshape(-1)
    row_pad = jnp.pad(row, (0, e_tc - n * k))

    xr = _gather_rows(x_pad, row_pad)

    blk = rx * k
    h1, st1 = pl.pallas_call(
        functools.partial(_econv_a_body, n, rx),
        grid=(nch,),
        in_specs=[
            pl.BlockSpec((blk, cin), lambda i: (i, 0)),
            pl.BlockSpec((rx, cin), lambda i: (i, 0)),
            pl.BlockSpec((cout, 2 * cin), lambda i: (0, 0)),
            pl.BlockSpec((1, cout), lambda i: (0, 0)),
        ],
        out_specs=[
            pl.BlockSpec((blk, cout), lambda i: (i, 0)),
            pl.BlockSpec((1, 2, cout), lambda i: (i, 0, 0)),
        ],
        out_shape=[
            jax.ShapeDtypeStruct((e_tc, cout), jnp.float32),
            jax.ShapeDtypeStruct((nch, 2, cout), jnp.float32),
        ],
    )(xr, x_pad, p['w1'], p['b1'][None, :])
    e_real = jnp.float32(n * k)
    s = jnp.sum(st1, axis=0)
    m1 = (s[0] / e_real)[None, :]
    v1 = (s[1] / e_real - (s[0] / e_real) ** 2)[None, :]

    h2, st2 = pl.pallas_call(
        functools.partial(_econv_b_body, n, rx),
        grid=(nch,),
        in_specs=[
            pl.BlockSpec((blk, cout), lambda i: (i, 0)),
            pl.BlockSpec((1, cout), lambda i: (0, 0)),
            pl.BlockSpec((1, cout), lambda i: (0, 0)),
            pl.BlockSpec((1, cout), lambda i: (0, 0)),
            pl.BlockSpec((1, cout), lambda i: (0, 0)),
            pl.BlockSpec((cout, cout), lambda i: (0, 0)),
            pl.BlockSpec((1, cout), lambda i: (0, 0)),
        ],
        out_specs=[
            pl.BlockSpec((blk, cout), lambda i: (i, 0)),
            pl.BlockSpec((1, 2, cout), lambda i: (i, 0, 0)),
        ],
        out_shape=[
            jax.ShapeDtypeStruct((e_tc, cout), jnp.float32),
            jax.ShapeDtypeStruct((nch, 2, cout), jnp.float32),
        ],
    )(h1, m1, v1, p['g1'][None, :], p['be1'][None, :], p['w2'], p['b2'][None, :])
    s2 = jnp.sum(st2, axis=0)
    m2 = (s2[0] / e_real)[None, :]
    v2 = (s2[1] / e_real - (s2[0] / e_real) ** 2)[None, :]

    h3 = pl.pallas_call(
        _econv_c_body,
        grid=(nch,),
        in_specs=[
            pl.BlockSpec((blk, cout), lambda i: (i, 0)),
            pl.BlockSpec((1, cout), lambda i: (0, 0)),
            pl.BlockSpec((1, cout), lambda i: (0, 0)),
            pl.BlockSpec((1, cout), lambda i: (0, 0)),
            pl.BlockSpec((1, cout), lambda i: (0, 0)),
        ],
        out_specs=pl.BlockSpec((blk, cout), lambda i: (i, 0)),
        out_shape=jax.ShapeDtypeStruct((e_tc, cout), jnp.float32),
    )(h2, m2, v2, p['g2'][None, :], p['be2'][None, :])

    agg = jnp.zeros((n, cout), jnp.float32).at[row].max(h3[:n * k])
    return agg


def _lbr_body(x_ref, w_ref, b_ref, g_ref, be_ref, o_ref):
    h = lax.dot_general(x_ref[...].astype(jnp.bfloat16), w_ref[...].astype(jnp.bfloat16),
                        (((1,), (1,)), ((), ())),
                        preferred_element_type=jnp.float32) + b_ref[...]
    m = jnp.mean(h, axis=0)
    v = jnp.mean((h - m) ** 2, axis=0)
    hb = (h - m) / jnp.sqrt(v + EPS_BN) * g_ref[...] + be_ref[...]
    o_ref[...] = jnp.maximum(hb, 0.0)


def _lbr_pallas(p, x):
    n = x.shape[0]
    cout = p['w'].shape[0]
    return pl.pallas_call(
        _lbr_body,
        out_shape=jax.ShapeDtypeStruct((n, cout), jnp.float32),
    )(x, p['w'], p['b'][None, :], p['g'][None, :], p['be'][None, :])


def _wsum_body(xg_ref, w_ref, o_ref):
    r = w_ref.shape[0]
    c = xg_ref.shape[1]
    xg = xg_ref[...].reshape(r, 3, c)
    w = w_ref[...]
    o_ref[...] = ((xg[:, 0, :] * w[:, 0:1] + xg[:, 1, :] * w[:, 1:2])
                  + xg[:, 2, :] * w[:, 2:3])


def _knn_interpolate_pallas(x, pos_x, pos_y, k=3):
    idx = _knn_topk(lax.stop_gradient(pos_y), lax.stop_gradient(pos_x), k, False)
    diff = pos_y[:, None, :] - pos_x[idx]
    sqd = jnp.sum(diff * diff, axis=-1)
    w = 1.0 / jnp.clip(sqd, 1e-16, None)
    w = w / jnp.sum(w, axis=1, keepdims=True)
    n = pos_y.shape[0]
    c = x.shape[1]
    xg = _gather_rows(x, idx.reshape(-1))
    rb = 512
    n_pad = ((n + rb - 1) // rb) * rb
    xg = jnp.pad(xg, ((0, (n_pad - n) * 3), (0, 0)))
    w = jnp.pad(w, ((0, n_pad - n), (0, 0)))
    out = pl.pallas_call(
        _wsum_body,
        grid=(n_pad // rb,),
        in_specs=[
            pl.BlockSpec((3 * rb, c), lambda i: (i, 0)),
            pl.BlockSpec((rb, 3), lambda i: (i, 0)),
        ],
        out_specs=pl.BlockSpec((rb, c), lambda i: (i, 0)),
        out_shape=jax.ShapeDtypeStruct((n_pad, c), jnp.float32),
    )(xg, w)
    return out[:n]


def _batchnorm(x, g, b):
    m = jnp.mean(x, axis=0)
    v = jnp.var(x, axis=0)
    return (x - m) / jnp.sqrt(v + EPS_BN) * g + b


def _knn_graph(pos, k, chunk=2500):
    n = pos.shape[0]
    sq = jnp.sum(pos * pos, axis=1)
    nbrs = []
    for i in range(0, n, chunk):
        q = pos[i:i + chunk]
        d = jnp.sum(q * q, axis=1, keepdims=True) - 2.0 * (q @ pos.T) + sq[None, :]
        gidx = jnp.arange(i, i + q.shape[0])
        d = jnp.where(jnp.arange(n)[None, :] == gidx[:, None], jnp.inf, d)
        _, idx = jax.lax.top_k(-d, k)
        nbrs.append(idx)
    nbr = jnp.concatenate(nbrs, axis=0)
    row = nbr.reshape(-1)
    col = jnp.repeat(jnp.arange(n), k)
    return row, col


def _knn_query(query, keys, k, chunk=2500):
    ksq = jnp.sum(keys * keys, axis=1)
    out = []
    for i in range(0, query.shape[0], chunk):
        q = query[i:i + chunk]
        d = jnp.sum(q * q, axis=1, keepdims=True) - 2.0 * (q @ keys.T) + ksq[None, :]
        _, idx = jax.lax.top_k(-d, k)
        out.append(idx)
    return jnp.concatenate(out, axis=0)


def _edge_conv(p, x, pos, k=K_NN):
    nbr = _knn_topk(lax.stop_gradient(pos), lax.stop_gradient(pos), k, True)
    row = nbr.reshape(-1)
    col = jnp.repeat(jnp.arange(pos.shape[0]), k)
    feat = jnp.concatenate([x[row], x[col] - x[row]], axis=1)
    h = feat @ p['w1'].T + p['b1']
    h = jax.nn.relu(_batchnorm(h, p['g1'], p['be1']))
    h = h @ p['w2'].T + p['b2']
    h = jax.nn.relu(_batchnorm(h, p['g2'], p['be2']))
    agg = jnp.zeros((x.shape[0], h.shape[1]), h.dtype).at[row].max(h)
    return agg


def _fps(pos, ratio):
    n = pos.shape[0]
    m = int(n * ratio)
    pos = lax.stop_gradient(pos)

    def body(i, state):
        dist, idxs = state
        last = idxs[i - 1]
        d = jnp.sum((pos - pos[last]) ** 2, axis=1)
        dist = jnp.minimum(dist, d)
        idxs = idxs.at[i].set(jnp.argmax(dist).astype(jnp.int32))
        return dist, idxs

    dist0 = jnp.full((n,), jnp.inf, jnp.float32)
    idxs0 = jnp.zeros((m,), jnp.int32)
    _, idxs = lax.fori_loop(1, m, body, (dist0, idxs0))
    return idxs


def _knn_interpolate(x, pos_x, pos_y, k=3):
    idx = _knn_topk(lax.stop_gradient(pos_y), lax.stop_gradient(pos_x), k, False)
    diff = pos_y[:, None, :] - pos_x[idx]
    sqd = jnp.sum(diff * diff, axis=-1)
    w = 1.0 / jnp.clip(sqd, 1e-16, None)
    w = w / jnp.sum(w, axis=1, keepdims=True)
    return jnp.sum(x[idx] * w[..., None], axis=1)


def _lbr(p, x):
    h = x @ p['w'].T + p['b']
    return jax.nn.relu(_batchnorm(h, p['g'], p['be']))


def kernel(x, pos, batch, params):
    x0, pos0 = x, pos
    x1 = _edge_conv_pallas(params['conv1'], x0, pos0)
    idx1 = _fps_pallas(pos0, 0.25)
    pos1, x1s = pos0[idx1], x1[idx1]
    x2 = _edge_conv_pallas(params['conv2'], x1s, pos1)
    idx2 = _fps_pallas(pos1, 0.25)
    pos2, x2s = pos1[idx2], x2[idx2]
    x3 = _edge_conv_pallas(params['conv3'], x2s, pos2)
    idx3 = _fps_pallas(pos2, 0.25)
    pos3, x3s = pos2[idx3], x3[idx3]
    x4 = _edge_conv_pallas(params['conv4'], x3s, pos3)
    up2 = _knn_interpolate_pallas(x4, pos3, pos2)
    d2 = _lbr_pallas(params['dec1'], jnp.concatenate([up2, x3], axis=1))
    up1 = _knn_interpolate_pallas(d2, pos2, pos1)
    d1 = _lbr_pallas(params['dec2'], jnp.concatenate([up1, x2], axis=1))
    up0 = _knn_interpolate_pallas(d1, pos1, pos0)
    d0 = _lbr_pallas(params['dec3'], jnp.concatenate([up0, x1], axis=1))
    final = jnp.concatenate([d0, x0], axis=1)
    return _head_pallas(final, params['head1'], params['head2'])


# speedup vs baseline: 1.0183x; 1.0014x over previous
import functools

import jax
import jax.numpy as jnp
from jax import lax
from jax.experimental import pallas as pl
from jax.experimental.pallas import tpu as pltpu
from jax.experimental.pallas import tpu_sc as plsc

N_POINTS = 10000
NUM_FEATURES = 128
NUM_CLASSES = 16
K_NN = 16
EPS_BN = 1e-5


def _head_body(x_ref, w1_ref, b1_ref, g1_ref, be1_ref, w2_ref, b2_ref, o_ref):
    x = x_ref[...]
    h = lax.dot_general(x.astype(jnp.bfloat16), w1_ref[...].astype(jnp.bfloat16),
                        (((1,), (1,)), ((), ())),
                        preferred_element_type=jnp.float32) + b1_ref[...]
    m = jnp.mean(h, axis=0)
    v = jnp.mean((h - m) ** 2, axis=0)
    h = (h - m) / jnp.sqrt(v + EPS_BN) * g1_ref[...] + be1_ref[...]
    h = jnp.maximum(h, 0.0)
    o = lax.dot_general(h.astype(jnp.bfloat16), w2_ref[...].astype(jnp.bfloat16),
                        (((1,), (1,)), ((), ())),
                        preferred_element_type=jnp.float32) + b2_ref[...]
    shifted = o - jnp.max(o, axis=-1, keepdims=True)
    o_ref[...] = shifted - jnp.log(jnp.sum(jnp.exp(shifted), axis=-1, keepdims=True))


def _head_pallas(xcat, p1, p2):
    n = xcat.shape[0]
    return pl.pallas_call(
        _head_body,
        out_shape=jax.ShapeDtypeStruct((n, NUM_CLASSES), jnp.float32),
    )(xcat, p1['w'], p1['b'], p1['g'], p1['be'], p2['w'], p2['b'])


def _fps_body(m, n, r, planes_ref, prow_ref, out_ref):
    fio = (lax.broadcasted_iota(jnp.int32, (r, 128), 0) * 128
           + lax.broadcasted_iota(jnp.int32, (r, 128), 1))
    px = planes_ref[0]
    py = planes_ref[1]
    pz = planes_ref[2]
    dist0 = jnp.where(fio < n, jnp.inf, -jnp.inf).astype(jnp.float32)
    out_ref[pl.ds(0, 1), :] = jnp.zeros((1, 1), jnp.int32)

    def step(i, carry):
        dist, last = carry
        prow = prow_ref[pl.ds(last, 1), :]
        lx, ly, lz = prow[0, 0], prow[0, 1], prow[0, 2]
        dx, dy, dz = px - lx, py - ly, pz - lz
        d = (dx * dx + dy * dy) + dz * dz
        dist = jnp.minimum(dist, d)
        mx = jnp.max(dist)
        idx = jnp.min(jnp.where(dist == mx, fio, jnp.int32(2**30)))
        out_ref[pl.ds(i, 1), :] = jnp.full((1, 1), idx, jnp.int32)
        return dist, idx

    lax.fori_loop(1, m, step, (dist0, jnp.int32(0)), unroll=False)


def _fps_pallas(pos, ratio):
    n = pos.shape[0]
    m = int(n * ratio)
    p = ((n + 127) // 128) * 128
    r = p // 128
    planes = jnp.pad(pos, ((0, p - n), (0, 0))).T.reshape(3, r, 128)
    prow = jnp.pad(pos, ((0, p - n), (0, 125)))
    out = pl.pallas_call(
        functools.partial(_fps_body, m, n, r),
        out_shape=jax.ShapeDtypeStruct((m, 1), jnp.int32),
    )(planes, prow)
    return out[:, 0]


def _knn_body(n_q, n_k, p, ch, k, excl, planes_ref, q_ref, out_ref):
    i = pl.program_id(0)
    px, py, pz = planes_ref[0], planes_ref[1], planes_ref[2]
    qx, qy, qz = q_ref[:, 0:1], q_ref[:, 1:2], q_ref[:, 2:3]

    def _rb(v):
        return v.astype(jnp.bfloat16).astype(jnp.float32)

    dot = (_rb(qx) * _rb(px) + _rb(qy) * _rb(py)) + _rb(qz) * _rb(pz)
    qsq = (qx * qx + qy * qy) + qz * qz
    sq = (px * px + py * py) + pz * pz
    d = (qsq - 2.0 * dot) + sq
    colio = lax.broadcasted_iota(jnp.int32, (ch, p), 1)
    if excl:
        rowio = lax.broadcasted_iota(jnp.int32, (ch, p), 0) + i * ch
        d = jnp.where(colio == rowio, jnp.inf, d)
    d = jnp.where(colio >= n_k, jnp.inf, d)
    outs = []
    for _ in range(k):
        mn = jnp.min(d, axis=1, keepdims=True)
        sel = jnp.min(jnp.where(d == mn, colio, jnp.int32(2**30)), axis=1)
        outs.append(sel[:, None])
        d = jnp.where(colio == sel[:, None], jnp.inf, d)
    out_ref[...] = jnp.concatenate(outs, axis=1)


def _knn_topk(query, keys, k, exclude_self):
    n_q, n_k = query.shape[0], keys.shape[0]
    p = ((n_k + 127) // 128) * 128
    qpad = ((n_q + 7) // 8) * 8
    ch = qpad if qpad <= 512 else 512
    qpad = ((n_q + ch - 1) // ch) * ch
    planes = jnp.pad(keys, ((0, p - n_k), (0, 0))).T.reshape(3, 1, p)
    qrows = jnp.pad(query, ((0, qpad - n_q), (0, 0)))
    out = pl.pallas_call(
        functools.partial(_knn_body, n_q, n_k, p, ch, k, exclude_self),
        grid=(qpad // ch,),
        in_specs=[
            pl.BlockSpec((3, 1, p), lambda i: (0, 0, 0)),
            pl.BlockSpec((ch, 3), lambda i: (i, 0)),
        ],
        out_specs=pl.BlockSpec((ch, k), lambda i: (i, 0)),
        out_shape=jax.ShapeDtypeStruct((qpad, k), jnp.int32),
    )(planes, qrows)
    return out[:n_q]


def _sc_gather(table, idx):
    v_rows, d = table.shape
    b = idx.shape[0]
    ch = 128
    chunks = b // (32 * ch)
    mesh = plsc.VectorSubcoreMesh(core_axis_name="c", subcore_axis_name="s")

    @functools.partial(
        pl.kernel, mesh=mesh,
        out_type=jax.ShapeDtypeStruct((b, d), jnp.float32),
        scratch_types=[
            pltpu.VMEM((ch,), jnp.int32),
            pltpu.VMEM((ch, d), jnp.float32),
            pltpu.SemaphoreType.DMA,
        ],
    )
    def k(table_hbm, idx_hbm, out_hbm, idx_v, rows_v, sem):
        wid = lax.axis_index("s") * 2 + lax.axis_index("c")

        def body(c, carry):
            base = (wid * chunks + c) * ch
            pltpu.sync_copy(idx_hbm.at[pl.ds(base, ch)], idx_v)
            pltpu.async_copy(table_hbm.at[idx_v], rows_v, sem).wait()
            pltpu.sync_copy(rows_v, out_hbm.at[pl.ds(base, ch)])
            return carry

        lax.fori_loop(0, chunks, body, 0)

    return k(table, idx)


def _gather_rows(table, idx, d_pad=None):
    v_rows, d = table.shape
    b = idx.shape[0]
    dp = d_pad if d_pad is not None else ((d + 127) // 128) * 128
    bp = ((b + 4095) // 4096) * 4096
    t = table if dp == d else jnp.pad(table, ((0, 0), (0, dp - d)))
    ix = jnp.pad(idx, (0, bp - b)) if bp != b else idx
    out = _sc_gather(t, ix)
    return out[:b, :d]


def _econv_a_body(n, rx, xr_ref, xc_ref, w1_ref, b1_ref, h1_ref, st_ref):
    i = pl.program_id(0)
    cin = xc_ref.shape[1]
    xr = xr_ref[...]
    xc = xc_ref[...]
    xcr = jnp.broadcast_to(xc[:, None, :], (rx, K_NN, cin)).reshape(rx * K_NN, cin)
    feat = jnp.concatenate([xr, xcr - xr], axis=1)
    h = lax.dot_general(feat.astype(jnp.bfloat16), w1_ref[...].astype(jnp.bfloat16),
                        (((1,), (1,)), ((), ())),
                        preferred_element_type=jnp.float32) + b1_ref[...]
    h1_ref[...] = h
    rowio = lax.broadcasted_iota(jnp.int32, (rx * K_NN, 1), 0) // K_NN + i * rx
    hm = jnp.where(rowio < n, h, 0.0)
    st_ref[...] = jnp.concatenate(
        [jnp.sum(hm, axis=0)[None, None, :], jnp.sum(hm * hm, axis=0)[None, None, :]],
        axis=1)


def _econv_b_body(n, rx, h1_ref, m_ref, v_ref, g_ref, be_ref, w2_ref, b2_ref,
                  h2_ref, st_ref):
    i = pl.program_id(0)
    h = h1_ref[...]
    hb = (h - m_ref[...]) / jnp.sqrt(v_ref[...] + EPS_BN) * g_ref[...] + be_ref[...]
    gact = jnp.maximum(hb, 0.0)
    h2 = lax.dot_general(gact.astype(jnp.bfloat16), w2_ref[...].astype(jnp.bfloat16),
                         (((1,), (1,)), ((), ())),
                         preferred_element_type=jnp.float32) + b2_ref[...]
    h2_ref[...] = h2
    rowio = lax.broadcasted_iota(jnp.int32, (rx * K_NN, 1), 0) // K_NN + i * rx
    hm = jnp.where(rowio < n, h2, 0.0)
    st_ref[...] = jnp.concatenate(
        [jnp.sum(hm, axis=0)[None, None, :], jnp.sum(hm * hm, axis=0)[None, None, :]],
        axis=1)


def _econv_c_body(h2_ref, m_ref, v_ref, g_ref, be_ref, o_ref):
    h = h2_ref[...]
    hb = (h - m_ref[...]) / jnp.sqrt(v_ref[...] + EPS_BN) * g_ref[...] + be_ref[...]
    o_ref[...] = jnp.maximum(hb, 0.0)


def _edge_conv_pallas(p, x, pos, k=K_NN):
    n, cin = x.shape
    cout = p['w1'].shape[0]
    nbr = _knn_topk(lax.stop_gradient(pos), lax.stop_gradient(pos), k, True)

    if n == 10000:
        rx = 400
    else:
        rx = min(512, ((n + 7) // 8) * 8)
    n_pad = ((n + rx - 1) // rx) * rx
    nch = n_pad // rx
    e_tc = n_pad * k
    x_pad = jnp.pad(x, ((0, n_pad - n), (0, 0))) if n_pad != n else x
    row = nbr.reshape(-1)
    row_pad = jnp.pad(row, (0, e_tc - n * k))

    xr = _gather_rows(x_pad, row_pad)

    blk = rx * k
    h1, st1 = pl.pallas_call(
        functools.partial(_econv_a_body, n, rx),
        grid=(nch,),
        in_specs=[
            pl.BlockSpec((blk, cin), lambda i: (i, 0)),
            pl.BlockSpec((rx, cin), lambda i: (i, 0)),
            pl.BlockSpec((cout, 2 * cin), lambda i: (0, 0)),
            pl.BlockSpec((1, cout), lambda i: (0, 0)),
        ],
        out_specs=[
            pl.BlockSpec((blk, cout), lambda i: (i, 0)),
            pl.BlockSpec((1, 2, cout), lambda i: (i, 0, 0)),
        ],
        out_shape=[
            jax.ShapeDtypeStruct((e_tc, cout), jnp.float32),
            jax.ShapeDtypeStruct((nch, 2, cout), jnp.float32),
        ],
    )(xr, x_pad, p['w1'], p['b1'][None, :])
    e_real = jnp.float32(n * k)
    s = jnp.sum(st1, axis=0)
    m1 = (s[0] / e_real)[None, :]
    v1 = (s[1] / e_real - (s[0] / e_real) ** 2)[None, :]

    h2, st2 = pl.pallas_call(
        functools.partial(_econv_b_body, n, rx),
        grid=(nch,),
        in_specs=[
            pl.BlockSpec((blk, cout), lambda i: (i, 0)),
            pl.BlockSpec((1, cout), lambda i: (0, 0)),
            pl.BlockSpec((1, cout), lambda i: (0, 0)),
            pl.BlockSpec((1, cout), lambda i: (0, 0)),
            pl.BlockSpec((1, cout), lambda i: (0, 0)),
            pl.BlockSpec((cout, cout), lambda i: (0, 0)),
            pl.BlockSpec((1, cout), lambda i: (0, 0)),
        ],
        out_specs=[
            pl.BlockSpec((blk, cout), lambda i: (i, 0)),
            pl.BlockSpec((1, 2, cout), lambda i: (i, 0, 0)),
        ],
        out_shape=[
            jax.ShapeDtypeStruct((e_tc, cout), jnp.float32),
            jax.ShapeDtypeStruct((nch, 2, cout), jnp.float32),
        ],
    )(h1, m1, v1, p['g1'][None, :], p['be1'][None, :], p['w2'], p['b2'][None, :])
    s2 = jnp.sum(st2, axis=0)
    m2 = (s2[0] / e_real)[None, :]
    v2 = (s2[1] / e_real - (s2[0] / e_real) ** 2)[None, :]

    h3 = pl.pallas_call(
        _econv_c_body,
        grid=(nch,),
        in_specs=[
            pl.BlockSpec((blk, cout), lambda i: (i, 0)),
            pl.BlockSpec((1, cout), lambda i: (0, 0)),
            pl.BlockSpec((1, cout), lambda i: (0, 0)),
            pl.BlockSpec((1, cout), lambda i: (0, 0)),
            pl.BlockSpec((1, cout), lambda i: (0, 0)),
        ],
        out_specs=pl.BlockSpec((blk, cout), lambda i: (i, 0)),
        out_shape=jax.ShapeDtypeStruct((e_tc, cout), jnp.float32),
    )(h2, m2, v2, p['g2'][None, :], p['be2'][None, :])

    agg = jnp.zeros((n, cout), jnp.float32).at[row].max(h3[:n * k])
    return agg


def _lbr_body(x_ref, w_ref, b_ref, g_ref, be_ref, o_ref):
    h = lax.dot_general(x_ref[...].astype(jnp.bfloat16), w_ref[...].astype(jnp.bfloat16),
                        (((1,), (1,)), ((), ())),
                        preferred_element_type=jnp.float32) + b_ref[...]
    m = jnp.mean(h, axis=0)
    v = jnp.mean((h - m) ** 2, axis=0)
    hb = (h - m) / jnp.sqrt(v + EPS_BN) * g_ref[...] + be_ref[...]
    o_ref[...] = jnp.maximum(hb, 0.0)


def _lbr_pallas(p, x):
    n = x.shape[0]
    cout = p['w'].shape[0]
    return pl.pallas_call(
        _lbr_body,
        out_shape=jax.ShapeDtypeStruct((n, cout), jnp.float32),
    )(x, p['w'], p['b'][None, :], p['g'][None, :], p['be'][None, :])


def _wsum_body(xg_ref, w_ref, o_ref):
    r = w_ref.shape[0]
    c = xg_ref.shape[1]
    xg = xg_ref[...].reshape(r, 3, c)
    w = w_ref[...]
    o_ref[...] = ((xg[:, 0, :] * w[:, 0:1] + xg[:, 1, :] * w[:, 1:2])
                  + xg[:, 2, :] * w[:, 2:3])


def _knn_interpolate_pallas(x, pos_x, pos_y, k=3):
    idx = _knn_topk(lax.stop_gradient(pos_y), lax.stop_gradient(pos_x), k, False)
    diff = pos_y[:, None, :] - pos_x[idx]
    sqd = jnp.sum(diff * diff, axis=-1)
    w = 1.0 / jnp.clip(sqd, 1e-16, None)
    w = w / jnp.sum(w, axis=1, keepdims=True)
    n = pos_y.shape[0]
    c = x.shape[1]
    xg = _gather_rows(x, idx.reshape(-1))
    rb = 512
    n_pad = ((n + rb - 1) // rb) * rb
    xg = jnp.pad(xg, ((0, (n_pad - n) * 3), (0, 0)))
    w = jnp.pad(w, ((0, n_pad - n), (0, 0)))
    out = pl.pallas_call(
        _wsum_body,
        grid=(n_pad // rb,),
        in_specs=[
            pl.BlockSpec((3 * rb, c), lambda i: (i, 0)),
            pl.BlockSpec((rb, 3), lambda i: (i, 0)),
        ],
        out_specs=pl.BlockSpec((rb, c), lambda i: (i, 0)),
        out_shape=jax.ShapeDtypeStruct((n_pad, c), jnp.float32),
    )(xg, w)
    return out[:n]


def kernel(x, pos, batch, params):
    x0, pos0 = x, pos
    x1 = _edge_conv_pallas(params['conv1'], x0, pos0)
    idx1 = _fps_pallas(pos0, 0.25)
    pos1, x1s = pos0[idx1], x1[idx1]
    x2 = _edge_conv_pallas(params['conv2'], x1s, pos1)
    idx2 = _fps_pallas(pos1, 0.25)
    pos2, x2s = pos1[idx2], x2[idx2]
    x3 = _edge_conv_pallas(params['conv3'], x2s, pos2)
    idx3 = _fps_pallas(pos2, 0.25)
    pos3, x3s = pos2[idx3], x3[idx3]
    x4 = _edge_conv_pallas(params['conv4'], x3s, pos3)
    up2 = _knn_interpolate_pallas(x4, pos3, pos2)
    d2 = _lbr_pallas(params['dec1'], jnp.concatenate([up2, x3], axis=1))
    up1 = _knn_interpolate_pallas(d2, pos2, pos1)
    d1 = _lbr_pallas(params['dec2'], jnp.concatenate([up1, x2], axis=1))
    up0 = _knn_interpolate_pallas(d1, pos1, pos0)
    d0 = _lbr_pallas(params['dec3'], jnp.concatenate([up0, x1], axis=1))
    final = jnp.concatenate([d0, x0], axis=1)
    return _head_pallas(final, params['head1'], params['head2'])
